# baseline (speedup 1.0000x reference)
import jax
import jax.numpy as jnp
from jax.experimental import pallas as pl
from jax.experimental.pallas import tpu as pltpu

_ROWS = 6144
_NBUF = 4


def _gate_body(x_hbm, idx_ref, w_ref, xbuf, sems):
    n = x_hbm.shape[0]
    nch = n // _ROWS

    def copy(j, slot):
        return pltpu.make_async_copy(
            x_hbm.at[pl.ds(j * _ROWS, _ROWS), :], xbuf.at[slot], sems.at[slot]
        )

    for s in range(_NBUF):
        copy(s, s).start()

    def loop(j, carry):
        slot = jax.lax.rem(j, _NBUF)
        copy(j, slot).wait()

        @pl.when(j + _NBUF < nch)
        def _():
            copy(j + _NBUF, slot).start()

        return carry

    jax.lax.fori_loop(0, nch, loop, 0)
    idx_ref[...] = jnp.zeros_like(idx_ref)
    w_ref[...] = jnp.zeros_like(w_ref) + xbuf[0, 0, 0]


def _route(x2):
    n = 32768
    return pl.pallas_call(
        _gate_body,
        in_specs=[pl.BlockSpec(memory_space=pl.ANY)],
        out_specs=[
            pl.BlockSpec(memory_space=pltpu.VMEM),
            pl.BlockSpec(memory_space=pltpu.VMEM),
        ],
        out_shape=[
            jax.ShapeDtypeStruct((n, 2), jnp.int32),
            jax.ShapeDtypeStruct((n, 2), jnp.float32),
        ],
        scratch_shapes=[
            pltpu.VMEM((_NBUF, _ROWS, 128), jnp.float32),
            pltpu.SemaphoreType.DMA((_NBUF,)),
        ],
    )(x2)


@jax.jit
def kernel(hidden_states, weight):
    x2 = hidden_states.reshape(-1, 128)
    topk_idx, topk_weight = _route(x2)
    return topk_idx, topk_weight


# speedup vs baseline: 2.1712x; 2.1712x over previous
import jax
import jax.numpy as jnp
from jax.experimental import pallas as pl
from jax.experimental.pallas import tpu as pltpu

_BLOCK = 8192
_SUB = 1024
_NBUF = 2
_NOUT = 4
_NE = 8


def _top2_block(x, wt):
    logits = jnp.dot(x, wt, preferred_element_type=jnp.float32)
    lane = jax.lax.broadcasted_iota(jnp.int32, logits.shape, 1)
    l1 = jnp.max(logits, axis=-1, keepdims=True)
    i1 = jnp.argmax(logits, axis=-1).astype(jnp.int32)[:, None]
    masked = jnp.where(lane == i1, -jnp.inf, logits)
    l2 = jnp.max(masked, axis=-1, keepdims=True)
    i2 = jnp.argmax(masked, axis=-1).astype(jnp.int32)[:, None]
    t = jnp.exp(l2 - l1)
    w1 = 1.0 / (1.0 + t)
    w2 = t * w1
    idx = jnp.concatenate([i1, i2], axis=1)
    w = jnp.concatenate([w1, w2], axis=1)
    return idx, w


def _gate_body(
    x_hbm, wt_ref, idx_hbm, w_hbm, xbuf, ibuf, wbuf, sems, isems, wsems
):
    n = x_hbm.shape[0]
    nch = n // _BLOCK
    nsub = _BLOCK // _SUB

    def copy(j, slot):
        return pltpu.make_async_copy(
            x_hbm.at[pl.ds(j * _BLOCK, _BLOCK), :], xbuf.at[slot], sems.at[slot]
        )

    def out_copies(g, oslot):
        return (
            pltpu.make_async_copy(
                ibuf.at[oslot], idx_hbm.at[pl.ds(g * _SUB, _SUB), :],
                isems.at[oslot],
            ),
            pltpu.make_async_copy(
                wbuf.at[oslot], w_hbm.at[pl.ds(g * _SUB, _SUB), :],
                wsems.at[oslot],
            ),
        )

    for s in range(_NBUF):
        copy(s, s).start()

    def loop(j, carry):
        slot = jax.lax.rem(j, _NBUF)
        copy(j, slot).wait()

        def sub(sb, c):
            g = j * nsub + sb
            oslot = jax.lax.rem(g, _NOUT)
            idx, w = _top2_block(xbuf[slot, pl.ds(sb * _SUB, _SUB), :], wt_ref[...])
            ic, wc = out_copies(g, oslot)

            @pl.when(g >= _NOUT)
            def _():
                pltpu.make_async_copy(
                    ibuf.at[oslot],
                    idx_hbm.at[pl.ds((g - _NOUT) * _SUB, _SUB), :],
                    isems.at[oslot],
                ).wait()
                pltpu.make_async_copy(
                    wbuf.at[oslot],
                    w_hbm.at[pl.ds((g - _NOUT) * _SUB, _SUB), :],
                    wsems.at[oslot],
                ).wait()

            ibuf[oslot] = idx
            wbuf[oslot] = w
            ic.start()
            wc.start()
            return c

        jax.lax.fori_loop(0, nsub, sub, 0)

        @pl.when(j + _NBUF < nch)
        def _():
            copy(j + _NBUF, slot).start()

        return carry

    jax.lax.fori_loop(0, nch, loop, 0)

    total = n // _SUB
    for k in range(_NOUT):
        g = total - _NOUT + k
        oslot = jax.lax.rem(g, _NOUT)
        pltpu.make_async_copy(
            ibuf.at[oslot], idx_hbm.at[pl.ds(g * _SUB, _SUB), :], isems.at[oslot]
        ).wait()
        pltpu.make_async_copy(
            wbuf.at[oslot], w_hbm.at[pl.ds(g * _SUB, _SUB), :], wsems.at[oslot]
        ).wait()


def _route(x, wt):
    n, h = x.shape
    return pl.pallas_call(
        _gate_body,
        in_specs=[
            pl.BlockSpec(memory_space=pl.ANY),
            pl.BlockSpec(memory_space=pltpu.VMEM),
        ],
        out_specs=[
            pl.BlockSpec(memory_space=pl.ANY),
            pl.BlockSpec(memory_space=pl.ANY),
        ],
        out_shape=[
            jax.ShapeDtypeStruct((n, 2), jnp.int32),
            jax.ShapeDtypeStruct((n, 2), jnp.float32),
        ],
        scratch_shapes=[
            pltpu.VMEM((_NBUF, _BLOCK, h), jnp.float32),
            pltpu.VMEM((_NOUT, _SUB, 2), jnp.int32),
            pltpu.VMEM((_NOUT, _SUB, 2), jnp.float32),
            pltpu.SemaphoreType.DMA((_NBUF,)),
            pltpu.SemaphoreType.DMA((_NOUT,)),
            pltpu.SemaphoreType.DMA((_NOUT,)),
        ],
        compiler_params=pltpu.CompilerParams(
            vmem_limit_bytes=62 * 1024 * 1024,
        ),
    )(x, wt)


@jax.jit
def kernel(hidden_states, weight):
    h = hidden_states.shape[-1]
    x = hidden_states.reshape(-1, h)
    topk_idx, topk_weight = _route(x, weight.T)
    return topk_idx, topk_weight


# speedup vs baseline: 2.3852x; 1.0986x over previous
import jax
import jax.numpy as jnp
from jax.experimental import pallas as pl
from jax.experimental.pallas import tpu as pltpu

_BLOCK = 512
_SUB = 512
_NBUF = 16
_NOUT = 4
_NE = 8


def _top2_block(x, wt):
    logits = jnp.dot(x, wt, preferred_element_type=jnp.float32)
    lane = jax.lax.broadcasted_iota(jnp.int32, logits.shape, 1)
    l1 = jnp.max(logits, axis=-1, keepdims=True)
    i1 = jnp.argmax(logits, axis=-1).astype(jnp.int32)[:, None]
    masked = jnp.where(lane == i1, -jnp.inf, logits)
    l2 = jnp.max(masked, axis=-1, keepdims=True)
    i2 = jnp.argmax(masked, axis=-1).astype(jnp.int32)[:, None]
    t = jnp.exp(l2 - l1)
    w1 = 1.0 / (1.0 + t)
    w2 = t * w1
    idx = jnp.concatenate([i1, i2], axis=1)
    w = jnp.concatenate([w1, w2], axis=1)
    return idx, w


def _gate_body(
    x_hbm, wt_ref, idx_hbm, w_hbm, xbuf, ibuf, wbuf, sems, isems, wsems
):
    n = x_hbm.shape[0]
    nch = n // _BLOCK
    nsub = _BLOCK // _SUB

    def copy(j, slot):
        return pltpu.make_async_copy(
            x_hbm.at[pl.ds(j * _BLOCK, _BLOCK), :], xbuf.at[slot], sems.at[slot]
        )

    def out_copies(g, oslot):
        return (
            pltpu.make_async_copy(
                ibuf.at[oslot], idx_hbm.at[pl.ds(g * _SUB, _SUB), :],
                isems.at[oslot],
            ),
            pltpu.make_async_copy(
                wbuf.at[oslot], w_hbm.at[pl.ds(g * _SUB, _SUB), :],
                wsems.at[oslot],
            ),
        )

    for s in range(_NBUF):
        copy(s, s).start()

    def loop(j, carry):
        slot = jax.lax.rem(j, _NBUF)
        copy(j, slot).wait()

        def sub(sb, c):
            g = j * nsub + sb
            oslot = jax.lax.rem(g, _NOUT)
            idx, w = _top2_block(xbuf[slot, pl.ds(sb * _SUB, _SUB), :], wt_ref[...])
            ic, wc = out_copies(g, oslot)

            @pl.when(g >= _NOUT)
            def _():
                pltpu.make_async_copy(
                    ibuf.at[oslot],
                    idx_hbm.at[pl.ds((g - _NOUT) * _SUB, _SUB), :],
                    isems.at[oslot],
                ).wait()
                pltpu.make_async_copy(
                    wbuf.at[oslot],
                    w_hbm.at[pl.ds((g - _NOUT) * _SUB, _SUB), :],
                    wsems.at[oslot],
                ).wait()

            ibuf[oslot] = idx
            wbuf[oslot] = w
            ic.start()
            wc.start()
            return c

        jax.lax.fori_loop(0, nsub, sub, 0)

        @pl.when(j + _NBUF < nch)
        def _():
            copy(j + _NBUF, slot).start()

        return carry

    jax.lax.fori_loop(0, nch, loop, 0)

    total = n // _SUB
    for k in range(_NOUT):
        g = total - _NOUT + k
        oslot = jax.lax.rem(g, _NOUT)
        pltpu.make_async_copy(
            ibuf.at[oslot], idx_hbm.at[pl.ds(g * _SUB, _SUB), :], isems.at[oslot]
        ).wait()
        pltpu.make_async_copy(
            wbuf.at[oslot], w_hbm.at[pl.ds(g * _SUB, _SUB), :], wsems.at[oslot]
        ).wait()


def _route(x, wt):
    n, h = x.shape
    return pl.pallas_call(
        _gate_body,
        in_specs=[
            pl.BlockSpec(memory_space=pl.ANY),
            pl.BlockSpec(memory_space=pltpu.VMEM),
        ],
        out_specs=[
            pl.BlockSpec(memory_space=pl.ANY),
            pl.BlockSpec(memory_space=pl.ANY),
        ],
        out_shape=[
            jax.ShapeDtypeStruct((n, 2), jnp.int32),
            jax.ShapeDtypeStruct((n, 2), jnp.float32),
        ],
        scratch_shapes=[
            pltpu.VMEM((_NBUF, _BLOCK, h), jnp.float32),
            pltpu.VMEM((_NOUT, _SUB, 2), jnp.int32),
            pltpu.VMEM((_NOUT, _SUB, 2), jnp.float32),
            pltpu.SemaphoreType.DMA((_NBUF,)),
            pltpu.SemaphoreType.DMA((_NOUT,)),
            pltpu.SemaphoreType.DMA((_NOUT,)),
        ],
        compiler_params=pltpu.CompilerParams(
            vmem_limit_bytes=62 * 1024 * 1024,
        ),
    )(x, wt)


@jax.jit
def kernel(hidden_states, weight):
    h = hidden_states.shape[-1]
    x = hidden_states.reshape(-1, h)
    topk_idx, topk_weight = _route(x, weight.T)
    return topk_idx, topk_weight
